# Initial kernel scaffold; baseline (speedup 1.0000x reference)
import functools

import jax
import jax.numpy as jnp
from jax import lax
from jax.experimental import pallas as pl
from jax.experimental.pallas import tpu as pltpu
from jax.experimental.pallas import tpu_sc as plsc

_CHUNK = 128


@functools.lru_cache(maxsize=None)
def _build_gather(n_rows, d):
    info = plsc.get_sparse_core_info()
    nc, ns = info.num_cores, info.num_subcores
    nw = nc * ns
    per_w = n_rows // nw
    n_chunks = per_w // _CHUNK
    assert per_w * nw == n_rows and n_chunks * _CHUNK == per_w

    mesh = plsc.VectorSubcoreMesh(core_axis_name="c", subcore_axis_name="s")

    @functools.partial(
        pl.kernel,
        mesh=mesh,
        out_type=jax.ShapeDtypeStruct((n_rows, d), jnp.float32),
        scratch_types=[
            pltpu.VMEM((n_chunks, _CHUNK), jnp.int32),
            pltpu.VMEM((_CHUNK, d), jnp.float32),
            pltpu.SemaphoreType.DMA,
        ],
    )
    def gather(table_hbm, idx_hbm, out_hbm, idx_v, rows_v, sem):
        wid = lax.axis_index("s") * nc + lax.axis_index("c")
        cbase = wid * n_chunks
        pltpu.sync_copy(idx_hbm.at[pl.ds(cbase, n_chunks)], idx_v)

        def step(j, carry):
            pltpu.async_copy(table_hbm.at[idx_v.at[j]], rows_v, sem).wait()
            pltpu.sync_copy(rows_v, out_hbm.at[pl.ds((cbase + j) * _CHUNK, _CHUNK)])
            return carry

        lax.fori_loop(0, n_chunks, step, 0)

    return gather


def kernel(input, table):
    b, l = input.shape
    _, d = table.shape
    n = b * l
    idx = input.reshape(n // _CHUNK, _CHUNK).astype(jnp.int32)
    out = _build_gather(n, d)(table, idx)
    return out.reshape(b, l, d)


# speedup vs baseline: 2.9702x; 2.9702x over previous
import functools

import jax
import jax.numpy as jnp
from jax import lax
from jax.experimental import pallas as pl
from jax.experimental.pallas import tpu as pltpu
from jax.experimental.pallas import tpu_sc as plsc

_CHUNK = 128


@functools.lru_cache(maxsize=None)
def _build_gather(n_rows, d):
    info = plsc.get_sparse_core_info()
    nc, ns = info.num_cores, info.num_subcores
    nw = nc * ns
    per_w = n_rows // nw
    n_chunks = per_w // _CHUNK
    assert per_w * nw == n_rows and n_chunks * _CHUNK == per_w

    mesh = plsc.VectorSubcoreMesh(core_axis_name="c", subcore_axis_name="s")

    @functools.partial(
        pl.kernel,
        mesh=mesh,
        out_type=jax.ShapeDtypeStruct((n_rows, d), jnp.float32),
        scratch_types=[
            pltpu.VMEM((n_chunks, _CHUNK), jnp.int32),
            pltpu.VMEM((_CHUNK, d), jnp.float32),
            pltpu.SemaphoreType.DMA,
        ],
    )
    def gather(table_hbm, idx_hbm, out_hbm, idx_v, rows_v, sem):
        wid = lax.axis_index("s") * nc + lax.axis_index("c")
        cbase = wid * n_chunks
        pltpu.sync_copy(idx_hbm.at[wid], idx_v)

        def step(j, carry):
            pltpu.async_copy(table_hbm.at[idx_v.at[j]], rows_v, sem).wait()
            pltpu.sync_copy(rows_v, out_hbm.at[pl.ds((cbase + j) * _CHUNK, _CHUNK)])
            return carry

        lax.fori_loop(0, n_chunks, step, 0)

    return gather


def kernel(input, table):
    b, l = input.shape
    _, d = table.shape
    n = b * l
    info = plsc.get_sparse_core_info()
    nw = info.num_cores * info.num_subcores
    idx = input.reshape(nw, n // (nw * _CHUNK), _CHUNK).astype(jnp.int32)
    out = _build_gather(n, d)(table, idx)
    return out.reshape(b, l, d)


# speedup vs baseline: 3.3087x; 1.1140x over previous
import functools

import jax
import jax.numpy as jnp
from jax import lax
from jax.experimental import pallas as pl
from jax.experimental.pallas import tpu as pltpu
from jax.experimental.pallas import tpu_sc as plsc

_CHUNK = 80
_NBUF = 4


@functools.lru_cache(maxsize=None)
def _build_gather(n_rows, d):
    info = plsc.get_sparse_core_info()
    nc, ns = info.num_cores, info.num_subcores
    nw = nc * ns
    per_w = n_rows // nw
    n_chunks = per_w // _CHUNK
    group = 2 * _NBUF
    n_super = n_chunks // group
    assert per_w * nw == n_rows
    assert n_chunks * _CHUNK == per_w
    assert n_super * group == n_chunks

    mesh = plsc.VectorSubcoreMesh(core_axis_name="c", subcore_axis_name="s")

    scratch = (
        [pltpu.VMEM((n_chunks, _CHUNK), jnp.int32)]
        + [pltpu.VMEM((_CHUNK, d), jnp.float32) for _ in range(group)]
        + [pltpu.SemaphoreType.DMA for _ in range(2 * group)]
    )

    @functools.partial(
        pl.kernel,
        mesh=mesh,
        out_type=jax.ShapeDtypeStruct((n_rows, d), jnp.float32),
        scratch_types=scratch,
    )
    def gather(table_hbm, idx_hbm, out_hbm, idx_v, *rest):
        bufs = rest[:group]
        gsems = rest[group:2 * group]
        ssems = rest[2 * group:]

        wid = lax.axis_index("s") * nc + lax.axis_index("c")
        cbase = wid * n_chunks
        pltpu.sync_copy(idx_hbm.at[wid], idx_v)

        def out_slice(c):
            return out_hbm.at[pl.ds((cbase + c) * _CHUNK, _CHUNK)]

        def body(s, carry):
            j0 = s * group
            hg = [
                pltpu.async_copy(table_hbm.at[idx_v.at[j0 + b]], bufs[b],
                                 gsems[b])
                for b in range(group)
            ]
            hs = []
            for b in range(group):
                hg[b].wait()
                hs.append(pltpu.async_copy(bufs[b], out_slice(j0 + b),
                                           ssems[b]))
            for h in hs:
                h.wait()
            return carry

        lax.fori_loop(0, n_super, body, 0)

    return gather


def kernel(input, table):
    b, l = input.shape
    _, d = table.shape
    n = b * l
    info = plsc.get_sparse_core_info()
    nw = info.num_cores * info.num_subcores
    idx = input.reshape(nw, n // (nw * _CHUNK), _CHUNK).astype(jnp.int32)
    out = _build_gather(n, d)(table, idx)
    return out.reshape(b, l, d)
